# Initial kernel scaffold; baseline (speedup 1.0000x reference)
import jax
import jax.numpy as jnp
from jax.experimental import pallas as pl


def kernel(q, k, embed_weight):
    raise NotImplementedError("write your pallas kernel here")


# speedup vs baseline: 17.9679x; 17.9679x over previous
import jax
import jax.numpy as jnp
from jax.experimental import pallas as pl
from jax.experimental.pallas import tpu as pltpu


def _copy_body(w_ref, o_ref):
    i = pl.program_id(0)
    offset = w_ref.shape[0] // 2 + w_ref.shape[0] % 2
    n = o_ref.shape[1]
    o_ref[0] = w_ref[pl.ds(offset - i, n), :]


def kernel(q, k, embed_weight):
    m = q.shape[0]
    n = k.shape[0]
    l, d = embed_weight.shape
    return pl.pallas_call(
        _copy_body,
        grid=(m,),
        in_specs=[pl.BlockSpec((l, d), lambda i: (0, 0))],
        out_specs=pl.BlockSpec((1, n, d), lambda i: (i, 0, 0)),
        out_shape=jax.ShapeDtypeStruct((m, n, d), embed_weight.dtype),
    )(embed_weight)
